# baseline (speedup 1.0000x reference)
import functools

import jax
import jax.numpy as jnp
from jax import lax
from jax.experimental import pallas as pl
from jax.experimental.pallas import tpu as pltpu
from jax.experimental.pallas import tpu_sc as plsc

MAX_DIST = 512
HEADS = 16
SEQ = 2048
TBL = 2 * MAX_DIST + 1
TROW = 1152
BLK = 16
DW = 2944
N_PASS = 8
N_K = 8


def _sc_body(bt_hbm, out_hbm, t_ref, d_refs, sems):
    wid = lax.axis_index("s") * 2 + lax.axis_index("c")
    h = wid // 2
    half = wid % 2
    ibase = half * (SEQ // 2)
    xbase = (1 - half) * (SEQ // 2)

    pltpu.sync_copy(bt_hbm.at[pl.ds(h * TROW, TROW)], t_ref)

    lane = lax.iota(jnp.int32, 16)

    def drain_one(sem):
        pltpu.make_async_copy(
            d_refs[0].at[:, pl.ds(0, SEQ)],
            out_hbm.at[0, pl.ds(0, BLK), :],
            sem,
        ).wait()

    for p in range(N_PASS):
        d_ref = d_refs[p % 2]
        sem = sems[p % 2]
        if p >= 2:
            for _ in range(N_K):
                drain_one(sem)

        base_p = (2544 - 16 * p) - xbase

        for b in range(BLK):
            c = base_p + b

            def bld(cx, _, d_ref=d_ref, b=b, c=c):
                x0 = cx * 16
                idx = jnp.clip((c - x0) - lane, 0, TBL - 1)
                d_ref[b, pl.ds(x0, 16)] = plsc.load_gather(t_ref, [idx])
                return 0

            if p < 2:
                lax.fori_loop(0, DW // 16, bld, 0, unroll=4)
            else:
                n_lo = (c - (TBL - 1)) // 16
                n_hi = (c + 15) // 16
                lax.fori_loop(n_lo, n_hi + 2, bld, 0)
        for k in range(N_K):
            blk = (7 - p) % 8 + 8 * k
            off = 896 - 128 * k
            pltpu.async_copy(
                d_ref.at[:, pl.ds(off, SEQ)],
                out_hbm.at[h, pl.ds(ibase + blk * BLK, BLK), :],
                sem,
            )
    for s in sems:
        for _ in range(N_K):
            drain_one(s)


@functools.partial(
    pl.kernel,
    out_type=jax.ShapeDtypeStruct((HEADS, SEQ, SEQ), jnp.float32),
    mesh=plsc.VectorSubcoreMesh(core_axis_name="c", subcore_axis_name="s"),
    scratch_types=[
        pltpu.VMEM((TROW,), jnp.float32),
        pltpu.VMEM((BLK, DW), jnp.float32),
        pltpu.VMEM((BLK, DW), jnp.float32),
        pltpu.SemaphoreType.DMA,
        pltpu.SemaphoreType.DMA,
    ],
    compiler_params=pltpu.CompilerParams(needs_layout_passes=False),
)
def _rel_pos_bias_sc(bt_hbm, out_hbm, t_ref, d_a, d_b, sem_a, sem_b):
    _sc_body(bt_hbm, out_hbm, t_ref, (d_a, d_b), (sem_a, sem_b))


def kernel(q_len, k_len, bias_table):
    bt = jnp.pad(bias_table.T, ((0, 0), (0, TROW - TBL))).reshape(-1)
    return _rel_pos_bias_sc(bt)


# speedup vs baseline: 1.0948x; 1.0948x over previous
import functools

import jax
import jax.numpy as jnp
from jax import lax
from jax.experimental import pallas as pl
from jax.experimental.pallas import tpu as pltpu
from jax.experimental.pallas import tpu_sc as plsc

MAX_DIST = 512
HEADS = 16
SEQ = 2048
TBL = 2 * MAX_DIST + 1
TROW = 1152
BLK = 16
DW = 2944
N_PASS = 8
N_K = 8


def _sc_body(bt_hbm, out_hbm, t_ref, d_refs, sems):
    wid = lax.axis_index("s") * 2 + lax.axis_index("c")
    h = wid // 2
    half = wid % 2
    ibase = half * (SEQ // 2)
    xbase = (1 - half) * (SEQ // 2)

    pltpu.sync_copy(bt_hbm.at[pl.ds(h * TROW, TROW)], t_ref)

    lane = lax.iota(jnp.int32, 16)

    def drain_one(sem):
        pltpu.make_async_copy(
            d_refs[0].at[:, pl.ds(0, SEQ)],
            out_hbm.at[0, pl.ds(0, BLK), :],
            sem,
        ).wait()

    for p in range(N_PASS):
        d_ref = d_refs[p % 2]
        sem = sems[p % 2]
        if p >= 2:
            for _ in range(N_K):
                drain_one(sem)

        base_p = (2544 - 16 * p) - xbase

        def row_build(b, _, d_ref=d_ref, base_p=base_p, p=p):
            c = base_p + b

            def bld(cx, _, d_ref=d_ref, b=b, c=c):
                x0 = cx * 16
                idx = jnp.clip((c - x0) - lane, 0, TBL - 1)
                d_ref[b, pl.ds(x0, 16)] = plsc.load_gather(t_ref, [idx])
                return 0

            if p < 2:
                lax.fori_loop(0, DW // 16, bld, 0)
            else:
                n_lo = (c - (TBL - 1)) // 16
                n_hi = (c + 15) // 16
                lax.fori_loop(n_lo, n_hi + 2, bld, 0)
            return 0

        lax.fori_loop(0, BLK, row_build, 0)
        for k in range(N_K):
            blk = (7 - p) % 8 + 8 * k
            off = 896 - 128 * k
            pltpu.async_copy(
                d_ref.at[:, pl.ds(off, SEQ)],
                out_hbm.at[h, pl.ds(ibase + blk * BLK, BLK), :],
                sem,
            )
    for s in sems:
        for _ in range(N_K):
            drain_one(s)


@functools.partial(
    pl.kernel,
    out_type=jax.ShapeDtypeStruct((HEADS, SEQ, SEQ), jnp.float32),
    mesh=plsc.VectorSubcoreMesh(core_axis_name="c", subcore_axis_name="s"),
    scratch_types=[
        pltpu.VMEM((TROW,), jnp.float32),
        pltpu.VMEM((BLK, DW), jnp.float32),
        pltpu.VMEM((BLK, DW), jnp.float32),
        pltpu.SemaphoreType.DMA,
        pltpu.SemaphoreType.DMA,
    ],
    compiler_params=pltpu.CompilerParams(needs_layout_passes=False),
)
def _rel_pos_bias_sc(bt_hbm, out_hbm, t_ref, d_a, d_b, sem_a, sem_b):
    _sc_body(bt_hbm, out_hbm, t_ref, (d_a, d_b), (sem_a, sem_b))


def kernel(q_len, k_len, bias_table):
    bt = jnp.pad(bias_table.T, ((0, 0), (0, TROW - TBL))).reshape(-1)
    return _rel_pos_bias_sc(bt)
